# baseline (speedup 1.0000x reference)
import jax
import jax.numpy as jnp
from jax.experimental import pallas as pl
from jax.experimental.pallas import tpu as pltpu

_N, _F, _H, _C = 100000, 128, 128, 64
_TILE = 2000


def _mlp_kernel(x_ref, w1_ref, b1_ref, w2_ref, b2_ref, o_ref):
    h = jax.lax.dot_general(
        x_ref[...], w1_ref[...],
        dimension_numbers=(((1,), (1,)), ((), ())),
        preferred_element_type=jnp.float32,
    )
    h = jnp.maximum(h + b1_ref[...], 0.0)
    o_ref[...] = jax.lax.dot_general(
        h, w2_ref[...],
        dimension_numbers=(((1,), (1,)), ((), ())),
        preferred_element_type=jnp.float32,
    ) + b2_ref[...]


def kernel(nodeblocks, x, W_in, b_in, W_out, b_out):
    b1 = b_in.reshape(1, _H)
    b2 = b_out.reshape(1, _C)
    return pl.pallas_call(
        _mlp_kernel,
        grid=(_N // _TILE,),
        in_specs=[
            pl.BlockSpec((_TILE, _F), lambda i: (i, 0)),
            pl.BlockSpec((_H, _F), lambda i: (0, 0)),
            pl.BlockSpec((1, _H), lambda i: (0, 0)),
            pl.BlockSpec((_C, _H), lambda i: (0, 0)),
            pl.BlockSpec((1, _C), lambda i: (0, 0)),
        ],
        out_specs=pl.BlockSpec((_TILE, _C), lambda i: (i, 0)),
        out_shape=jax.ShapeDtypeStruct((_N, _C), jnp.float32),
        compiler_params=pltpu.CompilerParams(
            dimension_semantics=("parallel",),
        ),
    )(x, W_in, b1, W_out, b2)


# speedup vs baseline: 1.3707x; 1.3707x over previous
import jax
import jax.numpy as jnp
from jax.experimental import pallas as pl
from jax.experimental.pallas import tpu as pltpu

_N, _F, _H, _C = 100000, 128, 128, 64
_TILE = 10000


def _mlp_kernel(x_ref, w1_ref, b1_ref, w2_ref, b2_ref, o_ref):
    h = jax.lax.dot_general(
        x_ref[...], w1_ref[...],
        dimension_numbers=(((1,), (1,)), ((), ())),
        preferred_element_type=jnp.float32,
    )
    h = jnp.maximum(h + b1_ref[...], 0.0)
    o_ref[...] = jax.lax.dot_general(
        h, w2_ref[...],
        dimension_numbers=(((1,), (1,)), ((), ())),
        preferred_element_type=jnp.float32,
    ) + b2_ref[...]


def kernel(nodeblocks, x, W_in, b_in, W_out, b_out):
    b1 = b_in.reshape(1, _H)
    b2 = b_out.reshape(1, _C)
    return pl.pallas_call(
        _mlp_kernel,
        grid=(_N // _TILE,),
        in_specs=[
            pl.BlockSpec((_TILE, _F), lambda i: (i, 0)),
            pl.BlockSpec((_H, _F), lambda i: (0, 0)),
            pl.BlockSpec((1, _H), lambda i: (0, 0)),
            pl.BlockSpec((_C, _H), lambda i: (0, 0)),
            pl.BlockSpec((1, _C), lambda i: (0, 0)),
        ],
        out_specs=pl.BlockSpec((_TILE, _C), lambda i: (i, 0)),
        out_shape=jax.ShapeDtypeStruct((_N, _C), jnp.float32),
        compiler_params=pltpu.CompilerParams(
            dimension_semantics=("parallel",),
        ),
    )(x, W_in, b1, W_out, b2)


# speedup vs baseline: 1.3862x; 1.0113x over previous
import jax
import jax.numpy as jnp
from jax.experimental import pallas as pl
from jax.experimental.pallas import tpu as pltpu

_N, _F, _H, _C = 100000, 128, 128, 64
_R = 2000
_S = _N // _R
_NBUF = 8


def _in_copy(x_hbm, xbuf, in_sem, c, b):
    return pltpu.make_async_copy(
        x_hbm.at[pl.ds(c * _R, _R)], xbuf.at[b], in_sem.at[b]
    )


def _out_copy(out_hbm, obuf, out_sem, c, b):
    return pltpu.make_async_copy(
        obuf.at[b], out_hbm.at[pl.ds(c * _R, _R)], out_sem.at[b]
    )


def _mlp_kernel(x_hbm, w1_ref, b1_ref, w2_ref, b2_ref, out_hbm,
                xbuf, obuf, in_sem, out_sem):
    for c in range(_NBUF):
        _in_copy(x_hbm, xbuf, in_sem, c, c).start()

    w1 = w1_ref[...]
    b1 = b1_ref[...]
    w2 = w2_ref[...]
    b2 = b2_ref[...]

    def step(c, carry):
        b = jax.lax.rem(c, _NBUF)
        _in_copy(x_hbm, xbuf, in_sem, c, b).wait()

        @pl.when(c >= _NBUF)
        def _():
            _out_copy(out_hbm, obuf, out_sem, c - _NBUF, b).wait()

        h = jax.lax.dot_general(
            xbuf[b], w1,
            dimension_numbers=(((1,), (1,)), ((), ())),
            preferred_element_type=jnp.float32,
        )
        h = jnp.maximum(h + b1, 0.0)
        obuf[b] = jax.lax.dot_general(
            h, w2,
            dimension_numbers=(((1,), (1,)), ((), ())),
            preferred_element_type=jnp.float32,
        ) + b2

        _out_copy(out_hbm, obuf, out_sem, c, b).start()

        @pl.when(c + _NBUF < _S)
        def _():
            _in_copy(x_hbm, xbuf, in_sem, c + _NBUF, b).start()

        return carry

    jax.lax.fori_loop(0, _S, step, 0)

    for k in range(_NBUF):
        c = _S - _NBUF + k
        _out_copy(out_hbm, obuf, out_sem, c, c % _NBUF).wait()


def kernel(nodeblocks, x, W_in, b_in, W_out, b_out):
    b1 = b_in.reshape(1, _H)
    b2 = b_out.reshape(1, _C)
    return pl.pallas_call(
        _mlp_kernel,
        in_specs=[
            pl.BlockSpec(memory_space=pltpu.MemorySpace.HBM),
            pl.BlockSpec(memory_space=pltpu.MemorySpace.VMEM),
            pl.BlockSpec(memory_space=pltpu.MemorySpace.VMEM),
            pl.BlockSpec(memory_space=pltpu.MemorySpace.VMEM),
            pl.BlockSpec(memory_space=pltpu.MemorySpace.VMEM),
        ],
        out_specs=pl.BlockSpec(memory_space=pltpu.MemorySpace.HBM),
        out_shape=jax.ShapeDtypeStruct((_N, _C), jnp.float32),
        scratch_shapes=[
            pltpu.VMEM((_NBUF, _R, _F), jnp.float32),
            pltpu.VMEM((_NBUF, _R, _C), jnp.float32),
            pltpu.SemaphoreType.DMA((_NBUF,)),
            pltpu.SemaphoreType.DMA((_NBUF,)),
        ],
    )(x, W_in, b1, W_out, b2)
